# baseline (speedup 1.0000x reference)
import jax
import jax.numpy as jnp
from jax.experimental import pallas as pl
from jax.experimental.pallas import tpu as pltpu

_BN_EPS = 1e-5
_BB = 8


def _gcn_body(x_ref, adjp_ref, W1_ref, b1_ref, Wb_ref, bb_ref, W2_ref, b2_ref,
              g_ref, beta_ref, out_ref):
    bf = jnp.bfloat16
    adjp = adjp_ref[...].astype(bf)
    W1 = W1_ref[...].astype(bf)
    Wb = Wb_ref[...].astype(bf)
    W2 = W2_ref[...].astype(bf)
    inv = 1.0 / (1.0 + _BN_EPS) ** 0.5
    gcol = g_ref[...] * inv
    bcol = beta_ref[...]

    def lanesum(s):
        k = 1
        while k < s.shape[1]:
            s = s + pltpu.roll(s, k, 1)
            k *= 2
        return s

    def layer(h, W, b):
        hp = jnp.dot(h, W, preferred_element_type=jnp.float32)
        t = jnp.dot(adjp, hp.astype(bf),
                    preferred_element_type=jnp.float32) + b
        n2 = lanesum(t * t)
        return t * jax.lax.rsqrt(n2 + 1e-24)

    def relu(h):
        return jnp.where(h > 0.0, h, 0.0)

    for i in range(_BB):
        h = layer(x_ref[i].astype(bf), W1, b1_ref[...])
        h = relu(h) * gcol + bcol
        h = layer(h.astype(bf), Wb, bb_ref[...])
        h = relu(h) * gcol + bcol
        h = layer(h.astype(bf), W2, b2_ref[...])
        out_ref[i] = h


_NCH = 16


def _head_body(h_ref, W3_ref, fc1b_ref, g1_ref, be1_ref, fc2W_ref,
               fc2b_ref, g2_ref, be2_ref, fc3W_ref, fc3b_ref, out_ref,
               acc_ref):
    s = pl.program_id(0)

    @pl.when(s == 0)
    def _init():
        acc_ref[...] = jnp.zeros_like(acc_ref)

    bf = jnp.bfloat16
    hb = h_ref[...].astype(bf)
    w = W3_ref[...].astype(bf)
    acc = acc_ref[...]
    for j in range(hb.shape[1]):
        acc = acc + jnp.dot(hb[:, j, :], w[j],
                            preferred_element_type=jnp.float32)
    acc_ref[...] = acc

    @pl.when(s == _NCH - 1)
    def _finish():
        inv = 1.0 / (1.0 + _BN_EPS) ** 0.5
        z = acc_ref[...] + fc1b_ref[...]
        z = jnp.where(z > 0.0, z, 0.0) * (g1_ref[...] * inv) + be1_ref[...]
        z = jnp.dot(z, fc2W_ref[...],
                    preferred_element_type=jnp.float32) + fc2b_ref[...]
        z = jnp.where(z > 0.0, z, 0.0) * (g2_ref[...] * inv) + be2_ref[...]
        out_ref[...] = jnp.dot(z, fc3W_ref[...],
                               preferred_element_type=jnp.float32) + fc3b_ref[...]


def kernel(x, adj, W1, b1, Wb, bb, W2, b2, bn_g, bn_b, fc1_W, fc1_b, bn1_g,
           bn1_b, fc2_W, fc2_b, bn2_g, bn2_b, fc3_W, fc3_b):
    B, N, F = x.shape
    H = W1.shape[1]
    E = W2.shape[1]
    NH = fc1_W.shape[1]
    L = fc3_W.shape[1]
    bf = jnp.bfloat16

    rep = lambda shape: pl.BlockSpec(shape, lambda b: (0,) * len(shape))

    h = pl.pallas_call(
        _gcn_body,
        grid=(B // _BB,),
        in_specs=[
            pl.BlockSpec((_BB, N, F), lambda b: (b, 0, 0)),
            rep((N, N)),
            rep((F, H)), rep((1, H)),
            rep((H, H)), rep((1, H)),
            rep((H, E)), rep((1, E)),
            rep((N, 1)), rep((N, 1)),
        ],
        out_specs=pl.BlockSpec((_BB, N, E), lambda b: (b, 0, 0)),
        out_shape=jax.ShapeDtypeStruct((B, N, E), jnp.float32),
        compiler_params=pltpu.CompilerParams(
            dimension_semantics=("arbitrary",)),
    )(x, adj + jnp.eye(N, dtype=adj.dtype), W1, b1.reshape(1, H),
      Wb, bb.reshape(1, H), W2, b2.reshape(1, E),
      bn_g.reshape(N, 1), bn_b.reshape(N, 1))

    CH = N // _NCH
    ypred = pl.pallas_call(
        _head_body,
        grid=(_NCH,),
        in_specs=[
            pl.BlockSpec((B, CH, E), lambda s: (0, s, 0)),
            pl.BlockSpec((CH, E, NH), lambda s: (s, 0, 0)),
            rep((1, NH)), rep((1, NH)), rep((1, NH)),
            rep((NH, NH)), rep((1, NH)), rep((1, NH)), rep((1, NH)),
            rep((NH, L)), rep((1, L)),
        ],
        out_specs=pl.BlockSpec((B, L), lambda s: (0, 0)),
        out_shape=jax.ShapeDtypeStruct((B, L), jnp.float32),
        scratch_shapes=[pltpu.VMEM((B, NH), jnp.float32)],
        compiler_params=pltpu.CompilerParams(
            dimension_semantics=("arbitrary",)),
    )(h, fc1_W.reshape(N, E, NH), fc1_b.reshape(1, NH),
      bn1_g.reshape(1, NH), bn1_b.reshape(1, NH), fc2_W,
      fc2_b.reshape(1, NH), bn2_g.reshape(1, NH), bn2_b.reshape(1, NH),
      fc3_W, fc3_b.reshape(1, L))

    return (ypred, h)


# speedup vs baseline: 2.5049x; 2.5049x over previous
import jax
import jax.numpy as jnp
from jax.experimental import pallas as pl
from jax.experimental.pallas import tpu as pltpu

_BN_EPS = 1e-5
_BB = 8


def _gcn_body(x_ref, adjp_ref, W1_ref, b1_ref, Wb_ref, bb_ref, W2_ref, b2_ref,
              g_ref, beta_ref, out_ref):
    bf = jnp.bfloat16
    adjp = adjp_ref[...].astype(bf)
    W1 = W1_ref[...].astype(bf)
    Wb = Wb_ref[...].astype(bf)
    W2 = W2_ref[...].astype(bf)
    inv = 1.0 / (1.0 + _BN_EPS) ** 0.5
    gcol = g_ref[...] * inv
    bcol = beta_ref[...]

    def norm(t):
        n2 = jnp.sum(t * t, axis=1, keepdims=True)
        return t * jax.lax.rsqrt(n2 + 1e-24)

    def relu(h):
        return jnp.where(h > 0.0, h, 0.0)

    def layer_all(hs, W, b):
        hps = [jnp.dot(h, W, preferred_element_type=jnp.float32) for h in hs]
        ts = [jnp.dot(adjp, hp.astype(bf),
                      preferred_element_type=jnp.float32) + b for hp in hps]
        return [norm(t) for t in ts]

    hs = [x_ref[i].astype(bf) for i in range(_BB)]
    hs = layer_all(hs, W1, b1_ref[...])
    hs = [relu(h) * gcol + bcol for h in hs]
    hs = layer_all([h.astype(bf) for h in hs], Wb, bb_ref[...])
    hs = [relu(h) * gcol + bcol for h in hs]
    hs = layer_all([h.astype(bf) for h in hs], W2, b2_ref[...])
    for i in range(_BB):
        out_ref[i] = hs[i]


_NCH = 16


def _head_body(h_ref, W3_ref, fc1b_ref, g1_ref, be1_ref, fc2W_ref,
               fc2b_ref, g2_ref, be2_ref, fc3W_ref, fc3b_ref, out_ref,
               acc_ref):
    s = pl.program_id(0)

    @pl.when(s == 0)
    def _init():
        acc_ref[...] = jnp.zeros_like(acc_ref)

    bf = jnp.bfloat16
    hb = h_ref[...].astype(bf)
    w = W3_ref[...].astype(bf)
    acc = acc_ref[...]
    for j in range(hb.shape[1]):
        acc = acc + jnp.dot(hb[:, j, :], w[j],
                            preferred_element_type=jnp.float32)
    acc_ref[...] = acc

    @pl.when(s == _NCH - 1)
    def _finish():
        inv = 1.0 / (1.0 + _BN_EPS) ** 0.5
        z = acc_ref[...] + fc1b_ref[...]
        z = jnp.where(z > 0.0, z, 0.0) * (g1_ref[...] * inv) + be1_ref[...]
        z = jnp.dot(z, fc2W_ref[...],
                    preferred_element_type=jnp.float32) + fc2b_ref[...]
        z = jnp.where(z > 0.0, z, 0.0) * (g2_ref[...] * inv) + be2_ref[...]
        out_ref[...] = jnp.dot(z, fc3W_ref[...],
                               preferred_element_type=jnp.float32) + fc3b_ref[...]


def kernel(x, adj, W1, b1, Wb, bb, W2, b2, bn_g, bn_b, fc1_W, fc1_b, bn1_g,
           bn1_b, fc2_W, fc2_b, bn2_g, bn2_b, fc3_W, fc3_b):
    B, N, F = x.shape
    H = W1.shape[1]
    E = W2.shape[1]
    NH = fc1_W.shape[1]
    L = fc3_W.shape[1]
    bf = jnp.bfloat16

    rep = lambda shape: pl.BlockSpec(shape, lambda b: (0,) * len(shape))

    h = pl.pallas_call(
        _gcn_body,
        grid=(B // _BB,),
        in_specs=[
            pl.BlockSpec((_BB, N, F), lambda b: (b, 0, 0)),
            rep((N, N)),
            rep((F, H)), rep((1, H)),
            rep((H, H)), rep((1, H)),
            rep((H, E)), rep((1, E)),
            rep((N, 1)), rep((N, 1)),
        ],
        out_specs=pl.BlockSpec((_BB, N, E), lambda b: (b, 0, 0)),
        out_shape=jax.ShapeDtypeStruct((B, N, E), jnp.float32),
        compiler_params=pltpu.CompilerParams(
            dimension_semantics=("arbitrary",)),
    )(x, adj + jnp.eye(N, dtype=adj.dtype), W1, b1.reshape(1, H),
      Wb, bb.reshape(1, H), W2, b2.reshape(1, E),
      bn_g.reshape(N, 1), bn_b.reshape(N, 1))

    CH = N // _NCH
    ypred = pl.pallas_call(
        _head_body,
        grid=(_NCH,),
        in_specs=[
            pl.BlockSpec((B, CH, E), lambda s: (0, s, 0)),
            pl.BlockSpec((CH, E, NH), lambda s: (s, 0, 0)),
            rep((1, NH)), rep((1, NH)), rep((1, NH)),
            rep((NH, NH)), rep((1, NH)), rep((1, NH)), rep((1, NH)),
            rep((NH, L)), rep((1, L)),
        ],
        out_specs=pl.BlockSpec((B, L), lambda s: (0, 0)),
        out_shape=jax.ShapeDtypeStruct((B, L), jnp.float32),
        scratch_shapes=[pltpu.VMEM((B, NH), jnp.float32)],
        compiler_params=pltpu.CompilerParams(
            dimension_semantics=("arbitrary",)),
    )(h, fc1_W.reshape(N, E, NH), fc1_b.reshape(1, NH),
      bn1_g.reshape(1, NH), bn1_b.reshape(1, NH), fc2_W,
      fc2_b.reshape(1, NH), bn2_g.reshape(1, NH), bn2_b.reshape(1, NH),
      fc3_W, fc3_b.reshape(1, L))

    return (ypred, h)


# speedup vs baseline: 2.5704x; 1.0261x over previous
import jax
import jax.numpy as jnp
from jax.experimental import pallas as pl
from jax.experimental.pallas import tpu as pltpu

_BN_EPS = 1e-5
_BB = 8


def _gcn_body(x_ref, adj_ref, W1_ref, b1_ref, Wb_ref, bb_ref, W2_ref, b2_ref,
              g_ref, beta_ref, out_ref):
    bf = jnp.bfloat16
    N = adj_ref.shape[0]
    row = jax.lax.broadcasted_iota(jnp.int32, (N, N), 0)
    col = jax.lax.broadcasted_iota(jnp.int32, (N, N), 1)
    adjp = (adj_ref[...] + jnp.where(row == col, 1.0, 0.0)).astype(bf)
    W1 = W1_ref[...].astype(bf)
    Wb = Wb_ref[...].astype(bf)
    W2 = W2_ref[...].astype(bf)
    inv = 1.0 / (1.0 + _BN_EPS) ** 0.5
    gcol = g_ref[...] * inv
    bcol = beta_ref[...]

    def norm(t):
        n2 = jnp.sum(t * t, axis=1, keepdims=True)
        return t * jax.lax.rsqrt(n2 + 1e-24)

    def relu(h):
        return jnp.where(h > 0.0, h, 0.0)

    def layer_all(hs, W, b):
        hps = [jnp.dot(h, W, preferred_element_type=jnp.float32) for h in hs]
        ts = [jnp.dot(adjp, hp.astype(bf),
                      preferred_element_type=jnp.float32) + b for hp in hps]
        return [norm(t) for t in ts]

    hs = [x_ref[i].astype(bf) for i in range(_BB)]
    hs = layer_all(hs, W1, b1_ref[...])
    hs = [relu(h) * gcol + bcol for h in hs]
    hs = layer_all([h.astype(bf) for h in hs], Wb, bb_ref[...])
    hs = [relu(h) * gcol + bcol for h in hs]
    hs = layer_all([h.astype(bf) for h in hs], W2, b2_ref[...])
    for i in range(_BB):
        out_ref[i] = hs[i]


_NCH = 16


def _head_body(h_ref, W3_ref, fc1b_ref, g1_ref, be1_ref, fc2W_ref,
               fc2b_ref, g2_ref, be2_ref, fc3W_ref, fc3b_ref, out_ref,
               acc_ref):
    s = pl.program_id(0)

    @pl.when(s == 0)
    def _init():
        acc_ref[...] = jnp.zeros_like(acc_ref)

    bf = jnp.bfloat16
    hb = h_ref[...].astype(bf)
    w = W3_ref[...].astype(bf)
    acc = acc_ref[...]
    for j in range(hb.shape[1]):
        acc = acc + jnp.dot(hb[:, j, :], w[j],
                            preferred_element_type=jnp.float32)
    acc_ref[...] = acc

    @pl.when(s == _NCH - 1)
    def _finish():
        inv = 1.0 / (1.0 + _BN_EPS) ** 0.5
        z = acc_ref[...] + fc1b_ref[...]
        z = jnp.where(z > 0.0, z, 0.0) * (g1_ref[...] * inv) + be1_ref[...]
        z = jnp.dot(z, fc2W_ref[...],
                    preferred_element_type=jnp.float32) + fc2b_ref[...]
        z = jnp.where(z > 0.0, z, 0.0) * (g2_ref[...] * inv) + be2_ref[...]
        out_ref[...] = jnp.dot(z, fc3W_ref[...],
                               preferred_element_type=jnp.float32) + fc3b_ref[...]


def kernel(x, adj, W1, b1, Wb, bb, W2, b2, bn_g, bn_b, fc1_W, fc1_b, bn1_g,
           bn1_b, fc2_W, fc2_b, bn2_g, bn2_b, fc3_W, fc3_b):
    B, N, F = x.shape
    H = W1.shape[1]
    E = W2.shape[1]
    NH = fc1_W.shape[1]
    L = fc3_W.shape[1]
    bf = jnp.bfloat16

    rep = lambda shape: pl.BlockSpec(shape, lambda b: (0,) * len(shape))

    h = pl.pallas_call(
        _gcn_body,
        grid=(B // _BB,),
        in_specs=[
            pl.BlockSpec((_BB, N, F), lambda b: (b, 0, 0)),
            rep((N, N)),
            rep((F, H)), rep((1, H)),
            rep((H, H)), rep((1, H)),
            rep((H, E)), rep((1, E)),
            rep((N, 1)), rep((N, 1)),
        ],
        out_specs=pl.BlockSpec((_BB, N, E), lambda b: (b, 0, 0)),
        out_shape=jax.ShapeDtypeStruct((B, N, E), jnp.float32),
        compiler_params=pltpu.CompilerParams(
            dimension_semantics=("arbitrary",)),
    )(x, adj, W1, b1.reshape(1, H),
      Wb, bb.reshape(1, H), W2, b2.reshape(1, E),
      bn_g.reshape(N, 1), bn_b.reshape(N, 1))

    CH = N // _NCH
    ypred = pl.pallas_call(
        _head_body,
        grid=(_NCH,),
        in_specs=[
            pl.BlockSpec((B, CH, E), lambda s: (0, s, 0)),
            pl.BlockSpec((CH, E, NH), lambda s: (s, 0, 0)),
            rep((1, NH)), rep((1, NH)), rep((1, NH)),
            rep((NH, NH)), rep((1, NH)), rep((1, NH)), rep((1, NH)),
            rep((NH, L)), rep((1, L)),
        ],
        out_specs=pl.BlockSpec((B, L), lambda s: (0, 0)),
        out_shape=jax.ShapeDtypeStruct((B, L), jnp.float32),
        scratch_shapes=[pltpu.VMEM((B, NH), jnp.float32)],
        compiler_params=pltpu.CompilerParams(
            dimension_semantics=("arbitrary",)),
    )(h, fc1_W.reshape(N, E, NH), fc1_b.reshape(1, NH),
      bn1_g.reshape(1, NH), bn1_b.reshape(1, NH), fc2_W,
      fc2_b.reshape(1, NH), bn2_g.reshape(1, NH), bn2_b.reshape(1, NH),
      fc3_W, fc3_b.reshape(1, L))

    return (ypred, h)


# speedup vs baseline: 2.6728x; 1.0399x over previous
import jax
import jax.numpy as jnp
from jax.experimental import pallas as pl
from jax.experimental.pallas import tpu as pltpu

_BN_EPS = 1e-5
_BB = 8


def _gcn_body(x_ref, adj_ref, W1_ref, b1_ref, Wb_ref, bb_ref, W2_ref, b2_ref,
              g_ref, beta_ref, out_ref):
    bf = jnp.bfloat16
    N = adj_ref.shape[0]
    row = jax.lax.broadcasted_iota(jnp.int32, (N, N), 0)
    col = jax.lax.broadcasted_iota(jnp.int32, (N, N), 1)
    adjp = (adj_ref[...] + jnp.where(row == col, 1.0, 0.0)).astype(bf)
    W1 = W1_ref[...].astype(bf)
    Wb = Wb_ref[...].astype(bf)
    W2 = W2_ref[...].astype(bf)
    inv = 1.0 / (1.0 + _BN_EPS) ** 0.5
    gcol = (g_ref[...] * inv).reshape(N, 1)
    bcol = beta_ref[...].reshape(N, 1)

    def norm(t):
        n2 = jnp.sum(t * t, axis=1, keepdims=True)
        return t * jax.lax.rsqrt(n2 + 1e-24)

    def relu(h):
        return jnp.where(h > 0.0, h, 0.0)

    def layer_all(hs, W, b):
        hps = [jnp.dot(h, W, preferred_element_type=jnp.float32) for h in hs]
        ts = [jnp.dot(adjp, hp.astype(bf),
                      preferred_element_type=jnp.float32) + b for hp in hps]
        return [norm(t) for t in ts]

    hs = [x_ref[i].astype(bf) for i in range(_BB)]
    hs = layer_all(hs, W1, b1_ref[...])
    hs = [relu(h) * gcol + bcol for h in hs]
    hs = layer_all([h.astype(bf) for h in hs], Wb, bb_ref[...])
    hs = [relu(h) * gcol + bcol for h in hs]
    hs = layer_all([h.astype(bf) for h in hs], W2, b2_ref[...])
    for i in range(_BB):
        out_ref[i] = hs[i]


_NCH = 16


def _head_body(h_ref, W3_ref, fc1b_ref, g1_ref, be1_ref, fc2W_ref,
               fc2b_ref, g2_ref, be2_ref, fc3W_ref, fc3b_ref, out_ref,
               acc_ref):
    s = pl.program_id(0)

    @pl.when(s == 0)
    def _init():
        acc_ref[...] = jnp.zeros_like(acc_ref)

    bf = jnp.bfloat16
    hb = h_ref[...].astype(bf)
    w = W3_ref[...].astype(bf)
    acc = acc_ref[...]
    for j in range(hb.shape[1]):
        acc = acc + jnp.dot(hb[:, j, :], w[j],
                            preferred_element_type=jnp.float32)
    acc_ref[...] = acc

    @pl.when(s == _NCH - 1)
    def _finish():
        inv = 1.0 / (1.0 + _BN_EPS) ** 0.5
        z = acc_ref[...] + fc1b_ref[...]
        z = jnp.where(z > 0.0, z, 0.0) * (g1_ref[...] * inv) + be1_ref[...]
        z = jnp.dot(z, fc2W_ref[...],
                    preferred_element_type=jnp.float32) + fc2b_ref[...]
        z = jnp.where(z > 0.0, z, 0.0) * (g2_ref[...] * inv) + be2_ref[...]
        out_ref[...] = jnp.dot(z, fc3W_ref[...],
                               preferred_element_type=jnp.float32) + fc3b_ref[...]


def kernel(x, adj, W1, b1, Wb, bb, W2, b2, bn_g, bn_b, fc1_W, fc1_b, bn1_g,
           bn1_b, fc2_W, fc2_b, bn2_g, bn2_b, fc3_W, fc3_b):
    B, N, F = x.shape
    H = W1.shape[1]
    E = W2.shape[1]
    NH = fc1_W.shape[1]
    L = fc3_W.shape[1]
    bf = jnp.bfloat16

    rep = lambda shape: pl.BlockSpec(shape, lambda b: (0,) * len(shape))

    h = pl.pallas_call(
        _gcn_body,
        grid=(B // _BB,),
        in_specs=[
            pl.BlockSpec((_BB, N, F), lambda b: (b, 0, 0)),
            rep((N, N)),
            rep((F, H)), rep((1, H)),
            rep((H, H)), rep((1, H)),
            rep((H, E)), rep((1, E)),
            rep((N,)), rep((N,)),
        ],
        out_specs=pl.BlockSpec((_BB, N, E), lambda b: (b, 0, 0)),
        out_shape=jax.ShapeDtypeStruct((B, N, E), jnp.float32),
        compiler_params=pltpu.CompilerParams(
            dimension_semantics=("arbitrary",)),
    )(x, adj, W1, b1.reshape(1, H),
      Wb, bb.reshape(1, H), W2, b2.reshape(1, E),
      bn_g, bn_b)

    CH = N // _NCH
    ypred = pl.pallas_call(
        _head_body,
        grid=(_NCH,),
        in_specs=[
            pl.BlockSpec((B, CH, E), lambda s: (0, s, 0)),
            pl.BlockSpec((CH, E, NH), lambda s: (s, 0, 0)),
            rep((1, NH)), rep((1, NH)), rep((1, NH)),
            rep((NH, NH)), rep((1, NH)), rep((1, NH)), rep((1, NH)),
            rep((NH, L)), rep((1, L)),
        ],
        out_specs=pl.BlockSpec((B, L), lambda s: (0, 0)),
        out_shape=jax.ShapeDtypeStruct((B, L), jnp.float32),
        scratch_shapes=[pltpu.VMEM((B, NH), jnp.float32)],
        compiler_params=pltpu.CompilerParams(
            dimension_semantics=("arbitrary",)),
    )(h, fc1_W.reshape(N, E, NH), fc1_b.reshape(1, NH),
      bn1_g.reshape(1, NH), bn1_b.reshape(1, NH), fc2_W,
      fc2_b.reshape(1, NH), bn2_g.reshape(1, NH), bn2_b.reshape(1, NH),
      fc3_W, fc3_b.reshape(1, L))

    return (ypred, h)


# speedup vs baseline: 2.6862x; 1.0050x over previous
import jax
import jax.numpy as jnp
from jax.experimental import pallas as pl
from jax.experimental.pallas import tpu as pltpu

_BN_EPS = 1e-5
_BB = 8
_GB = 8
_NCH = 16


def _body(x_ref, adj_ref, W1_ref, b1_ref, Wb_ref, bb_ref, W2_ref, b2_ref,
          g_ref, beta_ref, W3_ref, fc1b_ref, g1_ref, be1_ref, fc2W_ref,
          fc2b_ref, g2_ref, be2_ref, fc3W_ref, fc3b_ref,
          yp_ref, out_ref, hsc_ref, acc_ref):
    bf = jnp.bfloat16
    N = adj_ref.shape[0]
    s = pl.program_id(0)
    inv = 1.0 / (1.0 + _BN_EPS) ** 0.5

    @pl.when(s < _GB)
    def _conv_phase():
        row = jax.lax.broadcasted_iota(jnp.int32, (N, N), 0)
        col = jax.lax.broadcasted_iota(jnp.int32, (N, N), 1)
        adjp = (adj_ref[...] + jnp.where(row == col, 1.0, 0.0)).astype(bf)
        W1 = W1_ref[...].astype(bf)
        Wb = Wb_ref[...].astype(bf)
        W2 = W2_ref[...].astype(bf)
        gcol = (g_ref[...] * inv).reshape(N, 1)
        bcol = beta_ref[...].reshape(N, 1)

        def norm(t):
            n2 = jnp.sum(t * t, axis=1, keepdims=True)
            return t * jax.lax.rsqrt(n2 + 1e-24)

        def relu(h):
            return jnp.where(h > 0.0, h, 0.0)

        def layer_all(hs, W, b):
            hps = [jnp.dot(h, W, preferred_element_type=jnp.float32)
                   for h in hs]
            ts = [jnp.dot(adjp, hp.astype(bf),
                          preferred_element_type=jnp.float32) + b
                  for hp in hps]
            return [norm(t) for t in ts]

        hs = [x_ref[i].astype(bf) for i in range(_BB)]
        hs = layer_all(hs, W1, b1_ref[...])
        hs = [relu(h) * gcol + bcol for h in hs]
        hs = layer_all([h.astype(bf) for h in hs], Wb, bb_ref[...])
        hs = [relu(h) * gcol + bcol for h in hs]
        hs = layer_all([h.astype(bf) for h in hs], W2, b2_ref[...])
        for i in range(_BB):
            out_ref[i] = hs[i]
            hsc_ref[s * _BB + i] = hs[i].astype(bf)

    @pl.when(s == _GB)
    def _acc_init():
        acc_ref[...] = jnp.zeros_like(acc_ref)

    @pl.when(s >= _GB)
    def _head_phase():
        c = s - _GB
        CH = W3_ref.shape[0]
        w = W3_ref[...].astype(bf)
        hb = hsc_ref[:, pl.ds(c * CH, CH), :]
        acc = acc_ref[...]
        for j in range(CH):
            acc = acc + jnp.dot(hb[:, j, :], w[j],
                                preferred_element_type=jnp.float32)
        acc_ref[...] = acc

    @pl.when(s == _GB + _NCH - 1)
    def _finish():
        z = acc_ref[...] + fc1b_ref[...]
        z = jnp.where(z > 0.0, z, 0.0) * (g1_ref[...] * inv) + be1_ref[...]
        z = jnp.dot(z, fc2W_ref[...],
                    preferred_element_type=jnp.float32) + fc2b_ref[...]
        z = jnp.where(z > 0.0, z, 0.0) * (g2_ref[...] * inv) + be2_ref[...]
        yp_ref[...] = jnp.dot(z, fc3W_ref[...],
                              preferred_element_type=jnp.float32) + fc3b_ref[...]


def kernel(x, adj, W1, b1, Wb, bb, W2, b2, bn_g, bn_b, fc1_W, fc1_b, bn1_g,
           bn1_b, fc2_W, fc2_b, bn2_g, bn2_b, fc3_W, fc3_b):
    B, N, F = x.shape
    H = W1.shape[1]
    E = W2.shape[1]
    NH = fc1_W.shape[1]
    L = fc3_W.shape[1]
    CH = N // _NCH
    steps = _GB + _NCH

    rep = lambda shape: pl.BlockSpec(shape, lambda s: (0,) * len(shape))

    ypred, h = pl.pallas_call(
        _body,
        grid=(steps,),
        in_specs=[
            pl.BlockSpec((_BB, N, F),
                         lambda s: (jnp.minimum(s, _GB - 1), 0, 0)),
            rep((N, N)),
            rep((F, H)), rep((1, H)),
            rep((H, H)), rep((1, H)),
            rep((H, E)), rep((1, E)),
            rep((N,)), rep((N,)),
            pl.BlockSpec((CH, E, NH),
                         lambda s: (jnp.clip(s - _GB, 0, _NCH - 1), 0, 0)),
            rep((1, NH)), rep((1, NH)), rep((1, NH)),
            rep((NH, NH)), rep((1, NH)), rep((1, NH)), rep((1, NH)),
            rep((NH, L)), rep((1, L)),
        ],
        out_specs=[
            pl.BlockSpec((B, L), lambda s: (0, 0)),
            pl.BlockSpec((_BB, N, E),
                         lambda s: (jnp.minimum(s, _GB - 1), 0, 0)),
        ],
        out_shape=[
            jax.ShapeDtypeStruct((B, L), jnp.float32),
            jax.ShapeDtypeStruct((B, N, E), jnp.float32),
        ],
        scratch_shapes=[
            pltpu.VMEM((B, N, E), jnp.bfloat16),
            pltpu.VMEM((B, NH), jnp.float32),
        ],
        compiler_params=pltpu.CompilerParams(
            dimension_semantics=("arbitrary",)),
    )(x, adj, W1, b1.reshape(1, H), Wb, bb.reshape(1, H), W2,
      b2.reshape(1, E), bn_g, bn_b, fc1_W.reshape(N, E, NH),
      fc1_b.reshape(1, NH), bn1_g.reshape(1, NH), bn1_b.reshape(1, NH),
      fc2_W, fc2_b.reshape(1, NH), bn2_g.reshape(1, NH),
      bn2_b.reshape(1, NH), fc3_W, fc3_b.reshape(1, L))

    return (ypred, h)
